# Initial kernel scaffold; baseline (speedup 1.0000x reference)
import jax
import jax.numpy as jnp
from jax.experimental import pallas as pl


def kernel(x, adj, W1, b1, W2, b2):
    raise NotImplementedError("write your pallas kernel here")


# speedup vs baseline: 6.7972x; 6.7972x over previous
import functools
import jax
import jax.numpy as jnp
from jax import lax
from jax.experimental import pallas as pl
from jax.experimental.pallas import tpu as pltpu
from jax.experimental.pallas import tpu_sc as plsc

N_NODES = 10000
D_IN = 128
D_HID = 128
N_CLASS = 40
D_PAD = 48

N_SC = 2
N_TILES = 16
N_WORKERS = N_SC * N_TILES
CHUNK = 128
ROWS_PER_TILE = N_NODES // N_TILES


def _edge_aggregate(sup, src, dst, d):
    n_edges = src.shape[0]
    n_chunks = n_edges // CHUNK
    assert n_chunks * CHUNK == n_edges
    base_t = n_chunks // N_WORKERS
    extra = n_chunks - base_t * N_WORKERS

    mesh = plsc.VectorSubcoreMesh(core_axis_name="c", subcore_axis_name="s",
                                  num_cores=N_SC, num_subcores=N_TILES)

    def body(sup_hbm, src_hbm, dst_hbm, zeros_hbm, out_hbm,
             sidx_v, didx_v, rows_v, acc_sh, sem):
        c = lax.axis_index("c")
        s = lax.axis_index("s")
        wid = c * N_TILES + s
        pltpu.sync_copy(zeros_hbm,
                        acc_sh.at[pl.ds(s * ROWS_PER_TILE, ROWS_PER_TILE)])
        plsc.subcore_barrier()

        nt = base_t + jnp.where(wid < extra, 1, 0)

        def step(t, carry):
            base = (wid + N_WORKERS * t) * CHUNK
            pltpu.sync_copy(src_hbm.at[pl.ds(base, CHUNK)], sidx_v)
            pltpu.sync_copy(dst_hbm.at[pl.ds(base, CHUNK)], didx_v)
            pltpu.async_copy(sup_hbm.at[sidx_v], rows_v, sem).wait()
            pltpu.sync_copy(rows_v, acc_sh.at[didx_v], add=True)
            return carry

        lax.fori_loop(0, nt, step, 0)
        plsc.subcore_barrier()
        pltpu.sync_copy(acc_sh.at[pl.ds(s * ROWS_PER_TILE, ROWS_PER_TILE)],
                        out_hbm.at[c, s])

    kern = pl.kernel(
        body,
        out_type=jax.ShapeDtypeStruct((N_SC, N_TILES, ROWS_PER_TILE, d),
                                      jnp.float32),
        mesh=mesh,
        scratch_types=[
            pltpu.VMEM((CHUNK,), jnp.int32),
            pltpu.VMEM((CHUNK,), jnp.int32),
            pltpu.VMEM((CHUNK, d), jnp.float32),
            pltpu.VMEM_SHARED((N_NODES, d), jnp.float32),
            pltpu.SemaphoreType.DMA,
        ],
        compiler_params=pltpu.CompilerParams(use_tc_tiling_on_sc=False),
    )
    zeros = jnp.zeros((ROWS_PER_TILE, d), jnp.float32)
    out = kern(sup, src, dst, zeros)
    return out.reshape(N_SC, N_NODES, d)


def _matmul1(x, w):
    def body(x_ref, w_ref, o_ref):
        o_ref[...] = jnp.dot(x_ref[...], w_ref[...],
                             preferred_element_type=jnp.float32)

    return pl.pallas_call(
        body,
        grid=(10,),
        in_specs=[
            pl.BlockSpec((1000, D_IN), lambda i: (i, 0)),
            pl.BlockSpec((D_IN, D_HID), lambda i: (0, 0)),
        ],
        out_specs=pl.BlockSpec((1000, D_HID), lambda i: (i, 0)),
        out_shape=jax.ShapeDtypeStruct((N_NODES, D_HID), jnp.float32),
    )(x, w)


def _mid(p, b1, w2p):
    def body(p_ref, b_ref, w_ref, o_ref):
        h = jnp.maximum(p_ref[0] + p_ref[1] + b_ref[...], 0.0)
        o_ref[...] = jnp.dot(h, w_ref[...], preferred_element_type=jnp.float32)

    return pl.pallas_call(
        body,
        grid=(10,),
        in_specs=[
            pl.BlockSpec((N_SC, 1000, D_HID), lambda i: (0, i, 0)),
            pl.BlockSpec((1, D_HID), lambda i: (0, 0)),
            pl.BlockSpec((D_HID, D_PAD), lambda i: (0, 0)),
        ],
        out_specs=pl.BlockSpec((1000, D_PAD), lambda i: (i, 0)),
        out_shape=jax.ShapeDtypeStruct((N_NODES, D_PAD), jnp.float32),
    )(p, b1, w2p)


def _final(q, b2p):
    def body(q_ref, b_ref, o_ref):
        o_ref[...] = q_ref[0] + q_ref[1] + b_ref[...]

    return pl.pallas_call(
        body,
        grid=(10,),
        in_specs=[
            pl.BlockSpec((N_SC, 1000, D_PAD), lambda i: (0, i, 0)),
            pl.BlockSpec((1, D_PAD), lambda i: (0, 0)),
        ],
        out_specs=pl.BlockSpec((1000, D_PAD), lambda i: (i, 0)),
        out_shape=jax.ShapeDtypeStruct((N_NODES, D_PAD), jnp.float32),
    )(q, b2p)


@jax.jit
def kernel(x, adj, W1, b1, W2, b2):
    src = adj[0]
    dst = adj[1]
    w2p = jnp.pad(W2, ((0, 0), (0, D_PAD - N_CLASS)))
    b2p = jnp.pad(b2, (0, D_PAD - N_CLASS)).reshape(1, D_PAD)
    b1r = b1.reshape(1, D_HID)

    support1 = _matmul1(x, W1)
    p1 = _edge_aggregate(support1, src, dst, D_HID)
    support2 = _mid(p1, b1r, w2p)
    q = _edge_aggregate(support2, src, dst, D_PAD)
    out = _final(q, b2p)
    return out[:, :N_CLASS]
